# baseline (speedup 1.0000x reference)
import functools

import jax
import jax.numpy as jnp
from jax import lax
from jax.experimental import pallas as pl
from jax.experimental.pallas import tpu as pltpu
from jax.experimental.pallas import tpu_sc as plsc

D = 64
B = 16384
NC = 2
NS = 16
NW = NC * NS
BPW = B // NW
CH = 128
NCH = BPW // CH
L = 16


def _rsqrt16(x):
    i = lax.bitcast_convert_type(x, jnp.int32)
    i = jnp.int32(0x5F3759DF) - lax.shift_right_logical(i, 1)
    y = lax.bitcast_convert_type(i, jnp.float32)
    xh = x * 0.5
    for _ in range(3):
        y = y * (1.5 - xh * y * y)
    return y


def _bcast(v):
    return jnp.broadcast_to(v, (L,))


_mesh = plsc.VectorSubcoreMesh(core_axis_name="c", subcore_axis_name="s")


@functools.partial(
    pl.kernel,
    mesh=_mesh,
    compiler_params=pltpu.CompilerParams(
        needs_layout_passes=False, use_tc_tiling_on_sc=False),
    out_type=jax.ShapeDtypeStruct((B,), jnp.float32),
    scratch_types=[
        pltpu.VMEM((NCH, CH), jnp.int32),
        pltpu.VMEM((NCH, CH), jnp.int32),
        pltpu.VMEM((NCH, CH), jnp.int32),
        pltpu.VMEM((BPW, D), jnp.float32),
        pltpu.VMEM((BPW, D), jnp.float32),
        pltpu.VMEM((BPW, D), jnp.float32),
        pltpu.VMEM((BPW,), jnp.float32),
        pltpu.SemaphoreType.DMA,
    ],
)
def _sc_kernel(s_hbm, o_hbm, r_hbm, e_hbm, rt_hbm, out_hbm,
               si, oi, ri, se, oe, re_, ob, sem):
    wid = lax.axis_index("s") * NC + lax.axis_index("c")
    base = wid * BPW

    pltpu.sync_copy(s_hbm.at[wid], si)
    pltpu.sync_copy(o_hbm.at[wid], oi)
    pltpu.sync_copy(r_hbm.at[wid], ri)

    handles = []
    for j in range(NCH):
        dst = pl.ds(j * CH, CH)
        handles.append(pltpu.async_copy(e_hbm.at[si.at[j]], se.at[dst], sem))
        handles.append(pltpu.async_copy(e_hbm.at[oi.at[j]], oe.at[dst], sem))
        handles.append(pltpu.async_copy(rt_hbm.at[ri.at[j]], re_.at[dst], sem))
    for h in handles:
        h.wait()

    lanes = lax.iota(jnp.int32, L)
    cols = [jnp.full((L,), c, jnp.int32) for c in range(D)]

    def block(b, _):
        rows = b * L + lanes
        ss = jnp.zeros((L,), jnp.float32)
        so = jnp.zeros((L,), jnp.float32)
        sr = jnp.zeros((L,), jnp.float32)
        for c in range(D):
            vs = plsc.load_gather(se, [rows, cols[c]])
            vo = plsc.load_gather(oe, [rows, cols[c]])
            vr = plsc.load_gather(re_, [rows, cols[c]])
            ss = ss + vs * vs
            so = so + vo * vo
            sr = sr + vr * vr
        inv_s = _rsqrt16(jnp.maximum(ss, 1e-24))
        inv_o = _rsqrt16(jnp.maximum(so, 1e-24))
        inv_r = _rsqrt16(jnp.maximum(sr, 1e-24))
        score = jnp.zeros((L,), jnp.float32)
        for c in range(D):
            vs = plsc.load_gather(se, [rows, cols[c]])
            vo = plsc.load_gather(oe, [rows, cols[c]])
            vr = plsc.load_gather(re_, [rows, cols[c]])
            score = score + jnp.abs(vs * inv_s + vr * inv_r - vo * inv_o)
        ob[pl.ds(b * L, L)] = score
        return _

    lax.fori_loop(0, BPW // L, block, None)
    pltpu.sync_copy(ob, out_hbm.at[pl.ds(base, BPW)])


def kernel(s, r, o, e_table, r_table):
    s3 = s.astype(jnp.int32).reshape(NW, NCH, CH)
    o3 = o.astype(jnp.int32).reshape(NW, NCH, CH)
    r3 = r.astype(jnp.int32).reshape(NW, NCH, CH)
    return _sc_kernel(s3, o3, r3, e_table, r_table)


# speedup vs baseline: 1.0902x; 1.0902x over previous
import functools

import jax
import jax.numpy as jnp
from jax import lax
from jax.experimental import pallas as pl
from jax.experimental.pallas import tpu as pltpu
from jax.experimental.pallas import tpu_sc as plsc

D = 64
DP = 128
B = 16384
NC = 2
NS = 16
NW = NC * NS
BPW = B // NW
CH = 128
NCH = BPW // CH
L = 16


def _rsqrt16(x):
    i = lax.bitcast_convert_type(x, jnp.int32)
    i = jnp.int32(0x5F3759DF) - lax.shift_right_logical(i, 1)
    y = lax.bitcast_convert_type(i, jnp.float32)
    xh = x * 0.5
    for _ in range(3):
        y = y * (1.5 - xh * y * y)
    return y


_mesh = plsc.VectorSubcoreMesh(core_axis_name="c", subcore_axis_name="s")


@functools.partial(
    pl.kernel,
    mesh=_mesh,
    compiler_params=pltpu.CompilerParams(needs_layout_passes=False),
    out_type=jax.ShapeDtypeStruct((B,), jnp.float32),
    scratch_types=[
        pltpu.VMEM((NCH, CH), jnp.int32),
        pltpu.VMEM((NCH, CH), jnp.int32),
        pltpu.VMEM((NCH, CH), jnp.int32),
        pltpu.VMEM((CH, DP), jnp.float32),
        pltpu.VMEM((CH, DP), jnp.float32),
        pltpu.VMEM((CH, DP), jnp.float32),
        pltpu.VMEM((BPW,), jnp.float32),
        pltpu.SemaphoreType.DMA,
    ],
)
def _sc_kernel(s_hbm, o_hbm, r_hbm, e_hbm, rt_hbm, out_hbm,
               si, oi, ri, se, oe, re_, ob, sem):
    wid = lax.axis_index("s") * NC + lax.axis_index("c")
    base = wid * BPW

    pltpu.sync_copy(s_hbm.at[wid], si)
    pltpu.sync_copy(o_hbm.at[wid], oi)
    pltpu.sync_copy(r_hbm.at[wid], ri)

    lanes = lax.iota(jnp.int32, L)
    cols = [jnp.full((L,), c, jnp.int32) for c in range(D)]

    for j in range(NCH):
        hs = pltpu.async_copy(e_hbm.at[si.at[j]], se, sem)
        ho = pltpu.async_copy(e_hbm.at[oi.at[j]], oe, sem)
        hr = pltpu.async_copy(rt_hbm.at[ri.at[j]], re_, sem)
        hs.wait()
        ho.wait()
        hr.wait()

        def block(b, _):
            rows = b * L + lanes
            ss = jnp.zeros((L,), jnp.float32)
            so = jnp.zeros((L,), jnp.float32)
            sr = jnp.zeros((L,), jnp.float32)
            for c in range(D):
                vs = plsc.load_gather(se, [rows, cols[c]])
                vo = plsc.load_gather(oe, [rows, cols[c]])
                vr = plsc.load_gather(re_, [rows, cols[c]])
                ss = ss + vs * vs
                so = so + vo * vo
                sr = sr + vr * vr
            inv_s = _rsqrt16(jnp.maximum(ss, 1e-24))
            inv_o = _rsqrt16(jnp.maximum(so, 1e-24))
            inv_r = _rsqrt16(jnp.maximum(sr, 1e-24))
            score = jnp.zeros((L,), jnp.float32)
            for c in range(D):
                vs = plsc.load_gather(se, [rows, cols[c]])
                vo = plsc.load_gather(oe, [rows, cols[c]])
                vr = plsc.load_gather(re_, [rows, cols[c]])
                score = score + jnp.abs(vs * inv_s + vr * inv_r - vo * inv_o)
            ob[pl.ds(j * CH + b * L, L)] = score
            return _

        lax.fori_loop(0, CH // L, block, None)

    pltpu.sync_copy(ob, out_hbm.at[pl.ds(base, BPW)])


def kernel(s, r, o, e_table, r_table):
    s3 = s.astype(jnp.int32).reshape(NW, NCH, CH)
    o3 = o.astype(jnp.int32).reshape(NW, NCH, CH)
    r3 = r.astype(jnp.int32).reshape(NW, NCH, CH)
    ep = jnp.pad(e_table, ((0, 0), (0, DP - D)))
    rp = jnp.pad(r_table, ((0, 0), (0, DP - D)))
    return _sc_kernel(s3, o3, r3, ep, rp)


# speedup vs baseline: 1.5333x; 1.4065x over previous
import functools

import jax
import jax.numpy as jnp
from jax import lax
from jax.experimental import pallas as pl
from jax.experimental.pallas import tpu as pltpu
from jax.experimental.pallas import tpu_sc as plsc

D = 64
B = 16384
NC = 2
NS = 16
NW = NC * NS
BPW = B // NW
L = 16
CHE = 16
NCHE = BPW // CHE
GR = CHE * 8


def _rsqrt16(x):
    i = lax.bitcast_convert_type(x, jnp.int32)
    i = jnp.int32(0x5F3759DF) - lax.shift_right_logical(i, 1)
    y = lax.bitcast_convert_type(i, jnp.float32)
    xh = x * 0.5
    for _ in range(3):
        y = y * (1.5 - xh * y * y)
    return y


_mesh = plsc.VectorSubcoreMesh(core_axis_name="c", subcore_axis_name="s")


@functools.partial(
    pl.kernel,
    mesh=_mesh,
    compiler_params=pltpu.CompilerParams(needs_layout_passes=False),
    out_type=jax.ShapeDtypeStruct((B,), jnp.float32),
    scratch_types=[
        pltpu.VMEM((BPW,), jnp.int32),
        pltpu.VMEM((BPW,), jnp.int32),
        pltpu.VMEM((BPW,), jnp.int32),
        pltpu.VMEM((GR, D), jnp.float32),
        pltpu.VMEM((GR, D), jnp.float32),
        pltpu.VMEM((GR, D), jnp.float32),
        pltpu.VMEM((GR, D), jnp.float32),
        pltpu.VMEM((GR, D), jnp.float32),
        pltpu.VMEM((GR, D), jnp.float32),
        pltpu.VMEM((BPW,), jnp.float32),
        pltpu.SemaphoreType.DMA,
    ],
)
def _sc_kernel(s_hbm, o_hbm, r_hbm, e_hbm, rt_hbm, out_hbm,
               si, oi, ri, sa, oa, ra, sb, ob_, rb, res, sem):
    wid = lax.axis_index("s") * NC + lax.axis_index("c")
    base = wid * BPW

    pltpu.sync_copy(s_hbm.at[pl.ds(base, BPW)], si)
    pltpu.sync_copy(o_hbm.at[pl.ds(base, BPW)], oi)
    pltpu.sync_copy(r_hbm.at[pl.ds(base, BPW)], ri)

    lanes = lax.iota(jnp.int32, L)
    cols = [jnp.full((L,), c, jnp.int32) for c in range(D)]

    def issue(j, bufs):
        sd, od, rd = bufs
        evs = si[pl.ds(j * CHE, CHE)]
        evo = oi[pl.ds(j * CHE, CHE)]
        evr = ri[pl.ds(j * CHE, CHE)]
        for k in range(CHE):
            gs = pl.multiple_of((evs[k] >> 3) << 3, 8)
            go = pl.multiple_of((evo[k] >> 3) << 3, 8)
            gr = pl.multiple_of((evr[k] >> 3) << 3, 8)
            dst = pl.ds(k * 8, 8)
            pltpu.async_copy(e_hbm.at[pl.ds(gs, 8), :], sd.at[dst, :], sem)
            pltpu.async_copy(e_hbm.at[pl.ds(go, 8), :], od.at[dst, :], sem)
            pltpu.async_copy(rt_hbm.at[pl.ds(gr, 8), :], rd.at[dst, :], sem)

    def drain(bufs):
        for buf in bufs:
            pltpu.make_async_copy(e_hbm.at[pl.ds(0, GR), :], buf, sem).wait()

    def compute(j, bufs):
        sd, od, rd = bufs
        rows_s = lanes * 8 + (si[pl.ds(j * CHE, CHE)] & 7)
        rows_o = lanes * 8 + (oi[pl.ds(j * CHE, CHE)] & 7)
        rows_r = lanes * 8 + (ri[pl.ds(j * CHE, CHE)] & 7)
        ss = jnp.zeros((L,), jnp.float32)
        so = jnp.zeros((L,), jnp.float32)
        sr = jnp.zeros((L,), jnp.float32)
        for c in range(D):
            vs = plsc.load_gather(sd, [rows_s, cols[c]])
            vo = plsc.load_gather(od, [rows_o, cols[c]])
            vr = plsc.load_gather(rd, [rows_r, cols[c]])
            ss = ss + vs * vs
            so = so + vo * vo
            sr = sr + vr * vr
        inv_s = _rsqrt16(jnp.maximum(ss, 1e-24))
        inv_o = _rsqrt16(jnp.maximum(so, 1e-24))
        inv_r = _rsqrt16(jnp.maximum(sr, 1e-24))
        score = jnp.zeros((L,), jnp.float32)
        for c in range(D):
            vs = plsc.load_gather(sd, [rows_s, cols[c]])
            vo = plsc.load_gather(od, [rows_o, cols[c]])
            vr = plsc.load_gather(rd, [rows_r, cols[c]])
            score = score + jnp.abs(vs * inv_s + vr * inv_r - vo * inv_o)
        res[pl.ds(j * CHE, CHE)] = score

    bufs_a = (sa, oa, ra)
    bufs_b = (sb, ob_, rb)

    issue(jnp.int32(0), bufs_a)
    issue(jnp.int32(1), bufs_b)

    def step(t, _):
        ja = 2 * t
        drain(bufs_a)
        compute(ja, bufs_a)
        issue((ja + 2) & (NCHE - 1), bufs_a)
        drain(bufs_b)
        compute(ja + 1, bufs_b)
        issue((ja + 3) & (NCHE - 1), bufs_b)
        return _

    lax.fori_loop(0, NCHE // 2, step, None)
    drain(bufs_a)
    drain(bufs_b)

    pltpu.sync_copy(res, out_hbm.at[pl.ds(base, BPW)])


def kernel(s, r, o, e_table, r_table):
    return _sc_kernel(s.astype(jnp.int32), o.astype(jnp.int32),
                      r.astype(jnp.int32), e_table, r_table)


# speedup vs baseline: 2.0995x; 1.3692x over previous
import functools

import jax
import jax.numpy as jnp
from jax import lax
from jax.experimental import pallas as pl
from jax.experimental.pallas import tpu as pltpu
from jax.experimental.pallas import tpu_sc as plsc

D = 64
B = 16384
NC = 2
NS = 16
NW = NC * NS
BPW = B // NW
L = 16
CHE = 16
NCHE = BPW // CHE
GR = CHE * 8


def _rsqrt16(x):
    i = lax.bitcast_convert_type(x, jnp.int32)
    i = jnp.int32(0x5F3759DF) - lax.shift_right_logical(i, 1)
    y = lax.bitcast_convert_type(i, jnp.float32)
    xh = x * 0.5
    for _ in range(3):
        y = y * (1.5 - xh * y * y)
    return y


_mesh = plsc.VectorSubcoreMesh(core_axis_name="c", subcore_axis_name="s")


@functools.partial(
    pl.kernel,
    mesh=_mesh,
    compiler_params=pltpu.CompilerParams(needs_layout_passes=False),
    out_type=jax.ShapeDtypeStruct((B,), jnp.float32),
    scratch_types=[
        pltpu.VMEM((BPW,), jnp.int32),
        pltpu.VMEM((BPW,), jnp.int32),
        pltpu.VMEM((BPW,), jnp.int32),
        pltpu.VMEM((GR, D), jnp.float32),
        pltpu.VMEM((GR, D), jnp.float32),
        pltpu.VMEM((GR, D), jnp.float32),
        pltpu.VMEM((GR, D), jnp.float32),
        pltpu.VMEM((GR, D), jnp.float32),
        pltpu.VMEM((GR, D), jnp.float32),
        pltpu.VMEM((BPW,), jnp.float32),
        pltpu.SemaphoreType.DMA,
    ],
)
def _sc_kernel(s_hbm, o_hbm, r_hbm, e_hbm, rt_hbm, out_hbm,
               si, oi, ri, sa, oa, ra, sb, ob_, rb, res, sem):
    wid = lax.axis_index("s") * NC + lax.axis_index("c")
    base = wid * BPW

    pltpu.sync_copy(s_hbm.at[pl.ds(base, BPW)], si)
    pltpu.sync_copy(o_hbm.at[pl.ds(base, BPW)], oi)
    pltpu.sync_copy(r_hbm.at[pl.ds(base, BPW)], ri)

    lanes = lax.iota(jnp.int32, L)
    cols = [jnp.full((L,), c, jnp.int32) for c in range(D)]

    def issue(j, bufs):
        sd, od, rd = bufs
        evs = si[pl.ds(j * CHE, CHE)]
        evo = oi[pl.ds(j * CHE, CHE)]
        evr = ri[pl.ds(j * CHE, CHE)]
        for k in range(CHE):
            gs = pl.multiple_of((evs[k] >> 3) << 3, 8)
            go = pl.multiple_of((evo[k] >> 3) << 3, 8)
            gr = pl.multiple_of((evr[k] >> 3) << 3, 8)
            dst = pl.ds(k * 8, 8)
            pltpu.async_copy(e_hbm.at[0, pl.ds(gs, 8), :], sd.at[dst, :], sem)
            pltpu.async_copy(e_hbm.at[0, pl.ds(go, 8), :], od.at[dst, :], sem)
            pltpu.async_copy(rt_hbm.at[0, pl.ds(gr, 8), :], rd.at[dst, :], sem)

    def drain(bufs):
        for buf in bufs:
            pltpu.make_async_copy(e_hbm.at[0, pl.ds(0, GR), :], buf, sem).wait()

    def compute(j, bufs):
        sd, od, rd = bufs
        rows_s = lanes * 8 + (si[pl.ds(j * CHE, CHE)] & 7)
        rows_o = lanes * 8 + (oi[pl.ds(j * CHE, CHE)] & 7)
        rows_r = lanes * 8 + (ri[pl.ds(j * CHE, CHE)] & 7)
        ss = jnp.zeros((L,), jnp.float32)
        so = jnp.zeros((L,), jnp.float32)
        sr = jnp.zeros((L,), jnp.float32)
        for c in range(D):
            vs = plsc.load_gather(sd, [rows_s, cols[c]])
            vo = plsc.load_gather(od, [rows_o, cols[c]])
            vr = plsc.load_gather(rd, [rows_r, cols[c]])
            ss = ss + vs * vs
            so = so + vo * vo
            sr = sr + vr * vr
        inv_s = _rsqrt16(jnp.maximum(ss, 1e-24))
        inv_o = _rsqrt16(jnp.maximum(so, 1e-24))
        inv_r = _rsqrt16(jnp.maximum(sr, 1e-24))
        score = jnp.zeros((L,), jnp.float32)
        for c in range(D):
            vs = plsc.load_gather(sd, [rows_s, cols[c]])
            vo = plsc.load_gather(od, [rows_o, cols[c]])
            vr = plsc.load_gather(rd, [rows_r, cols[c]])
            score = score + jnp.abs(vs * inv_s + vr * inv_r - vo * inv_o)
        res[pl.ds(j * CHE, CHE)] = score

    bufs_a = (sa, oa, ra)
    bufs_b = (sb, ob_, rb)

    issue(jnp.int32(0), bufs_a)
    issue(jnp.int32(1), bufs_b)

    def step(t, _):
        ja = 2 * t
        drain(bufs_a)
        compute(ja, bufs_a)
        issue((ja + 2) & (NCHE - 1), bufs_a)
        drain(bufs_b)
        compute(ja + 1, bufs_b)
        issue((ja + 3) & (NCHE - 1), bufs_b)
        return _

    lax.fori_loop(0, NCHE // 2, step, None)
    drain(bufs_a)
    drain(bufs_b)

    pltpu.sync_copy(res, out_hbm.at[pl.ds(base, BPW)])


def kernel(s, r, o, e_table, r_table):
    return _sc_kernel(s.astype(jnp.int32), o.astype(jnp.int32),
                      r.astype(jnp.int32), e_table.reshape(1, 1000000, D),
                      r_table.reshape(1, 1000, D))
